# baseline (speedup 1.0000x reference)
import jax
import jax.numpy as jnp
from jax import lax
from jax.experimental import pallas as pl
from jax.experimental.pallas import tpu as pltpu

VOCAB_EXCL_PAD = 5
HIDDEN_DIM = 32
OUTPUT_DIM = 25
MAX_COUNT = 16.0
SEQ = 200

BC = 4096
GROUP = 25


def _body(ids_ref, w1t_ref, w2_ref, b1_ref, b2_ref, g_ref, bt_ref, out_ref):
    b1c = jnp.transpose(b1_ref[...][None, :])
    b2c = jnp.transpose(b2_ref[...][None, :])
    gc = jnp.transpose(g_ref[...][None, :])
    btc = jnp.transpose(bt_ref[...][None, :])

    ids = ids_ref[...]
    packed = jnp.full(ids.shape, 1, jnp.int32) << ((ids << 2) + ids)
    wides = [jnp.zeros((1, ids.shape[1]), jnp.int32)
             for _ in range(VOCAB_EXCL_PAD)]
    for g0 in range(0, SEQ, GROUP):
        g1 = min(g0 + GROUP, SEQ)
        s = jnp.sum(packed[g0:g1], axis=0, keepdims=True)
        for v in range(VOCAB_EXCL_PAD):
            wides[v] = wides[v] + ((s >> (5 * (v + 1))) & 31)
    counts = jnp.concatenate(wides, axis=0).astype(jnp.float32)
    counts = counts * (1.0 / MAX_COUNT)

    h = lax.dot_general(w1t_ref[...], counts, (((0,), (0,)), ((), ())),
                        preferred_element_type=jnp.float32)
    h = jnp.maximum(h + b1c, 0.0)

    h2 = jnp.dot(w2_ref[...], h, preferred_element_type=jnp.float32)
    h2 = h2 + b2c
    mean = jnp.mean(h2, axis=0, keepdims=True)
    d = h2 - mean
    var = jnp.mean(d * d, axis=0, keepdims=True)
    hn = d * lax.rsqrt(var + 1e-5)
    hn = hn * gc + btc
    out_ref[...] = jnp.maximum(hn, 0.0)


@jax.jit
def kernel(padded_ids, W1, b1, W2, b2, gamma, beta):
    B = padded_ids.shape[0]
    ids_t = padded_ids.astype(jnp.int32).T
    w1t = W1.T

    out_t = pl.pallas_call(
        _body,
        grid=(B // BC,),
        in_specs=[
            pl.BlockSpec((SEQ, BC), lambda i: (0, i)),
            pl.BlockSpec((VOCAB_EXCL_PAD, HIDDEN_DIM), lambda i: (0, 0)),
            pl.BlockSpec((OUTPUT_DIM, HIDDEN_DIM), lambda i: (0, 0)),
            pl.BlockSpec((HIDDEN_DIM,), lambda i: (0,)),
            pl.BlockSpec((OUTPUT_DIM,), lambda i: (0,)),
            pl.BlockSpec((OUTPUT_DIM,), lambda i: (0,)),
            pl.BlockSpec((OUTPUT_DIM,), lambda i: (0,)),
        ],
        out_specs=pl.BlockSpec((OUTPUT_DIM, BC), lambda i: (0, i)),
        out_shape=jax.ShapeDtypeStruct((OUTPUT_DIM, B), jnp.float32),
    )(ids_t, w1t, W2, b1, b2, gamma, beta)
    return out_t.T
